# baseline (speedup 1.0000x reference)
import jax
import jax.numpy as jnp
import numpy as np
from jax import lax
from jax.experimental import pallas as pl
from jax.experimental.pallas import tpu as pltpu

Q, D, K = 676, 384, 65536
QP = 680
BK = 2048
NSTEP = K // BK
FMAP = 26
IMG = 224
KSIZE = 33
SIGMA = 4.0
BIGI = 2 ** 30


def _ka_body(patch_ref, lib_ref,
             mind2_ref, bidx_ref, mtest_ref, sstar_ref,
             pn_ref, mx_ref, am_ref, riota_ref, dt_ref):
    k = pl.program_id(0)

    @pl.when(k == 0)
    def _init():
        p = patch_ref[...]
        nrm = jnp.sqrt(jnp.sum(p * p, axis=1, keepdims=True))
        pn_ref[...] = p / jnp.clip(nrm, 1e-12)
        mx_ref[...] = jnp.full((8, QP), -jnp.inf, jnp.float32)
        am_ref[...] = jnp.zeros((8, QP), jnp.int32)
        riota_ref[...] = lax.broadcasted_iota(jnp.int32, (BK, QP), 0)

    def _reduce(buf):
        dt = dt_ref[buf]
        tmax = jnp.max(dt, axis=0)
        targ = jnp.min(jnp.where(dt == tmax[None, :], riota_ref[...], BIGI),
                       axis=0) + (k - 1) * BK
        cur = mx_ref[...]
        upd = tmax[None, :] > cur
        mx_ref[...] = jnp.where(upd, jnp.broadcast_to(tmax[None, :], (8, QP)),
                                cur)
        am_ref[...] = jnp.where(upd, jnp.broadcast_to(targ[None, :], (8, QP)),
                                am_ref[...])

    @pl.when((k > 0) & (k % 2 == 1))
    def _reduce0():
        _reduce(0)

    @pl.when((k > 0) & (k % 2 == 0))
    def _reduce1():
        _reduce(1)

    @pl.when(k < NSTEP)
    def _matmul():
        pn = pn_ref[...]
        lib = lib_ref[...]
        mm = lax.dot_general(lib, pn, (((1,), (1,)), ((), ())),
                             preferred_element_type=jnp.float32)

        @pl.when(k % 2 == 0)
        def _st0():
            dt_ref[0] = mm

        @pl.when(k % 2 == 1)
        def _st1():
            dt_ref[1] = mm

    @pl.when(k == NSTEP)
    def _finish():
        pn = pn_ref[...]
        a2 = jnp.sum(pn * pn, axis=1)
        md = a2 + 1.0 - 2.0 * mx_ref[0, :]
        mind2_ref[...] = jnp.broadcast_to(md[None, :], (8, QP))
        col = lax.broadcasted_iota(jnp.int32, (8, QP), 1)
        valid = col < Q
        minv = jnp.sqrt(jnp.clip(mind2_ref[...], 1e-12))
        mm = jnp.where(valid, minv, -jnp.inf)
        s_star = jnp.max(mm)
        s_idx = jnp.min(jnp.where(mm == s_star, col, BIGI))
        b_row = jnp.sum(jnp.where(col == s_idx, am_ref[...], 0), axis=1)
        bidx_ref[...] = jnp.broadcast_to(b_row[:, None], (8, 128))
        roh = lax.broadcasted_iota(jnp.int32, (QP, D), 0) == s_idx
        mt = jnp.sum(jnp.where(roh, pn, 0.0), axis=0)
        mtest_ref[...] = jnp.broadcast_to(mt[None, :], (8, D))
        sstar_ref[...] = jnp.full((8, 128), s_star, jnp.float32)


def _run_ka(patch_p, patch_lib, interpret=False):
    return pl.pallas_call(
        _ka_body,
        grid=(NSTEP + 1,),
        in_specs=[
            pl.BlockSpec((QP, D), lambda k: (0, 0)),
            pl.BlockSpec((BK, D), lambda k: (jnp.minimum(k, NSTEP - 1), 0)),
        ],
        out_specs=[
            pl.BlockSpec((8, QP), lambda k: (0, 0)),
            pl.BlockSpec((8, 128), lambda k: (0, 0)),
            pl.BlockSpec((8, D), lambda k: (0, 0)),
            pl.BlockSpec((8, 128), lambda k: (0, 0)),
        ],
        out_shape=[
            jax.ShapeDtypeStruct((8, QP), jnp.float32),
            jax.ShapeDtypeStruct((8, 128), jnp.int32),
            jax.ShapeDtypeStruct((8, D), jnp.float32),
            jax.ShapeDtypeStruct((8, 128), jnp.float32),
        ],
        scratch_shapes=[
            pltpu.VMEM((QP, D), jnp.float32),
            pltpu.VMEM((8, QP), jnp.float32),
            pltpu.VMEM((8, QP), jnp.int32),
            pltpu.VMEM((BK, QP), jnp.int32),
            pltpu.VMEM((2, BK, QP), jnp.float32),
        ],
        interpret=interpret,
    )(patch_p, patch_lib)


def _kb_body(bidx_ref, lib_ref, bblk_ref, mtest_ref, v26_ref, sstar_ref,
             rmat_ref, bmat_ref, s_ref, smap_ref, dd_ref):
    k = pl.program_id(0)
    b = bidx_ref[0]
    r = b - (b // 8) * 8
    i0 = lax.broadcasted_iota(jnp.int32, (8, D), 0)
    m_star = jnp.sum(jnp.where(i0 == r, bblk_ref[...], 0.0), axis=0)
    w = jnp.where(i0 == 0, jnp.broadcast_to(m_star[None, :], (8, D)),
                  jnp.where(i0 == 1, mtest_ref[...], 0.0))
    a2 = jnp.sum(w * w, axis=1)
    dots = lax.dot_general(w, lib_ref[...], (((1,), (1,)), ((), ())),
                           preferred_element_type=jnp.float32)
    off = pl.multiple_of(k * BK, BK)
    dd_ref[:, pl.ds(off, BK)] = a2[:, None] + 1.0 - 2.0 * dots

    @pl.when(k == NSTEP - 1)
    def _finish():
        dd = dd_ref[...]
        row = lax.broadcasted_iota(jnp.int32, (8, K), 0)
        col = lax.broadcasted_iota(jnp.int32, (8, K), 1)
        dm = jnp.where(row == 0, dd, jnp.inf)
        den = jnp.float32(0.0)
        for _ in range(5):
            m = jnp.min(dm)
            sel = jnp.min(jnp.where(dm == m, col, BIGI))
            dq = jnp.sum(jnp.where((col == sel) & (row == 1), dd, 0.0))
            den = den + jnp.exp(jnp.sqrt(jnp.clip(dq, 0.0)))
            dm = jnp.where(col == sel, jnp.inf, dm)

        dqb = jnp.sum(jnp.where((col == b) & (row == 1), dd, 0.0))
        num = jnp.exp(jnp.sqrt(jnp.clip(dqb, 0.0)))
        s_star = sstar_ref[0, 0]
        s_ref[...] = jnp.full((8, 128), (1.0 - num / den) * s_star,
                              jnp.float32)

        v = jnp.sqrt(jnp.clip(v26_ref[...], 1e-12))
        rm = rmat_ref[...]
        bm = bmat_ref[...]
        t1 = lax.dot_general(rm, v, (((1,), (0,)), ((), ())),
                             preferred_element_type=jnp.float32)
        t2 = lax.dot_general(t1, rm, (((1,), (1,)), ((), ())),
                             preferred_element_type=jnp.float32)
        t3 = lax.dot_general(bm, t2, (((1,), (0,)), ((), ())),
                             preferred_element_type=jnp.float32)
        smap_ref[...] = lax.dot_general(t3, bm, (((1,), (1,)), ((), ())),
                                        preferred_element_type=jnp.float32)


def _run_kb(bidx1, patch_lib, mtest, v26, sstar, rmat, bmat, interpret=False):
    grid_spec = pltpu.PrefetchScalarGridSpec(
        num_scalar_prefetch=1,
        grid=(NSTEP,),
        in_specs=[
            pl.BlockSpec((BK, D), lambda k, b: (k, 0)),
            pl.BlockSpec((8, D), lambda k, b: (b[0] // 8, 0)),
            pl.BlockSpec((8, D), lambda k, b: (0, 0)),
            pl.BlockSpec((FMAP, FMAP), lambda k, b: (0, 0)),
            pl.BlockSpec((8, 128), lambda k, b: (0, 0)),
            pl.BlockSpec((IMG, FMAP), lambda k, b: (0, 0)),
            pl.BlockSpec((IMG, IMG), lambda k, b: (0, 0)),
        ],
        out_specs=[
            pl.BlockSpec((8, 128), lambda k, b: (0, 0)),
            pl.BlockSpec((IMG, IMG), lambda k, b: (0, 0)),
        ],
        scratch_shapes=[pltpu.VMEM((8, K), jnp.float32)],
    )
    return pl.pallas_call(
        _kb_body,
        grid_spec=grid_spec,
        out_shape=[
            jax.ShapeDtypeStruct((8, 128), jnp.float32),
            jax.ShapeDtypeStruct((IMG, IMG), jnp.float32),
        ],
        interpret=interpret,
    )(bidx1, patch_lib, patch_lib, mtest, v26, sstar, rmat, bmat)


def _blur_matrix():
    ax = np.arange(KSIZE, dtype=np.float32) - (KSIZE // 2)
    g = np.exp(-(ax ** 2) / (2.0 * SIGMA ** 2))
    g = g / np.sum(g)
    pad = KSIZE // 2
    eye = np.eye(IMG, dtype=np.float32)
    xp = np.pad(eye, ((pad, pad), (0, 0)), mode="reflect")
    b = np.zeros((IMG, IMG), dtype=np.float32)
    for t in range(KSIZE):
        b += g[t] * xp[t:t + IMG, :]
    return b


_BMAT = _blur_matrix()


def _resize_matrix():
    return jax.image.resize(jnp.eye(FMAP, dtype=jnp.float32), (IMG, FMAP),
                            method="bilinear")


def _kernel_impl(patch, patch_lib, interpret=False):
    patch_p = jnp.zeros((QP, D), jnp.float32).at[:Q].set(patch)
    mind2, bidx, mtest, sstar = _run_ka(patch_p, patch_lib, interpret)
    bidx1 = bidx[0, 0].reshape(1).astype(jnp.int32)
    v26 = mind2[0, :Q].reshape(FMAP, FMAP)
    s_out, smap = _run_kb(bidx1, patch_lib, mtest, v26, sstar,
                          _resize_matrix(), jnp.asarray(_BMAT), interpret)
    return s_out[0, 0], smap.reshape(1, 1, IMG, IMG)


def kernel(patch, patch_lib):
    return _kernel_impl(patch, patch_lib, interpret=False)


# speedup vs baseline: 1.2039x; 1.2039x over previous
import jax
import jax.numpy as jnp
import numpy as np
from jax import lax
from jax.experimental import pallas as pl
from jax.experimental.pallas import tpu as pltpu

Q, D, K = 676, 384, 65536
QP = 680
BK = 2048
NSTEP = K // BK
FMAP = 26
IMG = 224
KSIZE = 33
SIGMA = 4.0
BIGI = 2 ** 30


def _ka_body(patch_ref, lib_ref,
             mind2_ref, bidx_ref, mtest_ref, sstar_ref,
             pn_ref, mx_ref, am_ref, riota_ref):
    k = pl.program_id(0)

    @pl.when(k == 0)
    def _init():
        p = patch_ref[...]
        nrm = jnp.sqrt(jnp.sum(p * p, axis=1, keepdims=True))
        pn_ref[...] = p / jnp.clip(nrm, 1e-12)
        mx_ref[...] = jnp.full((8, QP), -jnp.inf, jnp.float32)
        am_ref[...] = jnp.zeros((8, QP), jnp.int32)
        riota_ref[...] = lax.broadcasted_iota(jnp.int32, (BK, QP), 0)

    pn = pn_ref[...]
    lib = lib_ref[...]
    dt = lax.dot_general(lib, pn, (((1,), (1,)), ((), ())),
                         preferred_element_type=jnp.float32)
    tmax = jnp.max(dt, axis=0)
    targ = jnp.min(jnp.where(dt == tmax[None, :], riota_ref[...], BIGI),
                   axis=0) + k * BK
    cur = mx_ref[...]
    upd = tmax[None, :] > cur
    mx_ref[...] = jnp.where(upd, jnp.broadcast_to(tmax[None, :], (8, QP)), cur)
    am_ref[...] = jnp.where(upd, jnp.broadcast_to(targ[None, :], (8, QP)),
                            am_ref[...])

    @pl.when(k == NSTEP - 1)
    def _finish():
        a2 = jnp.sum(pn * pn, axis=1)
        md = a2 + 1.0 - 2.0 * mx_ref[0, :]
        mind2_ref[...] = jnp.broadcast_to(md[None, :], (8, QP))
        col = lax.broadcasted_iota(jnp.int32, (8, QP), 1)
        valid = col < Q
        minv = jnp.sqrt(jnp.clip(mind2_ref[...], 1e-12))
        mm = jnp.where(valid, minv, -jnp.inf)
        s_star = jnp.max(mm)
        s_idx = jnp.min(jnp.where(mm == s_star, col, BIGI))
        b_row = jnp.sum(jnp.where(col == s_idx, am_ref[...], 0), axis=1)
        bidx_ref[...] = jnp.broadcast_to(b_row[:, None], (8, 128))
        roh = lax.broadcasted_iota(jnp.int32, (QP, D), 0) == s_idx
        mt = jnp.sum(jnp.where(roh, pn, 0.0), axis=0)
        mtest_ref[...] = jnp.broadcast_to(mt[None, :], (8, D))
        sstar_ref[...] = jnp.full((8, 128), s_star, jnp.float32)


def _run_ka(patch_p, patch_lib, interpret=False):
    return pl.pallas_call(
        _ka_body,
        grid=(NSTEP,),
        in_specs=[
            pl.BlockSpec((QP, D), lambda k: (0, 0)),
            pl.BlockSpec((BK, D), lambda k: (k, 0)),
        ],
        out_specs=[
            pl.BlockSpec((8, QP), lambda k: (0, 0)),
            pl.BlockSpec((8, 128), lambda k: (0, 0)),
            pl.BlockSpec((8, D), lambda k: (0, 0)),
            pl.BlockSpec((8, 128), lambda k: (0, 0)),
        ],
        out_shape=[
            jax.ShapeDtypeStruct((8, QP), jnp.float32),
            jax.ShapeDtypeStruct((8, 128), jnp.int32),
            jax.ShapeDtypeStruct((8, D), jnp.float32),
            jax.ShapeDtypeStruct((8, 128), jnp.float32),
        ],
        scratch_shapes=[
            pltpu.VMEM((QP, D), jnp.float32),
            pltpu.VMEM((8, QP), jnp.float32),
            pltpu.VMEM((8, QP), jnp.int32),
            pltpu.VMEM((BK, QP), jnp.int32),
        ],
        interpret=interpret,
    )(patch_p, patch_lib)


KF = K // 8
BKF = BK // 8


def _kb_body(bidx_ref, lib_ref, bblk_ref, mtest_ref, v26_ref, sstar_ref,
             rmat_ref, bmat_ref, s_ref, smap_ref, dmf_ref, dqf_ref, jio_ref):
    k = pl.program_id(0)
    b = bidx_ref[0]
    r = b - (b // 8) * 8
    i0 = lax.broadcasted_iota(jnp.int32, (8, D), 0)
    m_star = jnp.sum(jnp.where(i0 == r, bblk_ref[...], 0.0), axis=0)
    w = jnp.where(i0 == 0, jnp.broadcast_to(m_star[None, :], (8, D)),
                  jnp.where(i0 == 1, mtest_ref[...], 0.0))
    a2 = jnp.sum(w * w, axis=1)
    dots = lax.dot_general(w, lib_ref[...], (((1,), (1,)), ((), ())),
                           preferred_element_type=jnp.float32)
    dd = a2[:, None] + 1.0 - 2.0 * dots
    off = pl.multiple_of(k * BKF, BKF)
    dmf_ref[:, pl.ds(off, BKF)] = dd[0, :].reshape(8, BKF)
    dqf_ref[:, pl.ds(off, BKF)] = dd[1, :].reshape(8, BKF)

    @pl.when(k == 0)
    def _initjio():
        row = lax.broadcasted_iota(jnp.int32, (8, KF), 0)
        col = lax.broadcasted_iota(jnp.int32, (8, KF), 1)
        jio_ref[...] = (col // BKF) * BK + row * BKF + col % BKF

    @pl.when(k == NSTEP - 1)
    def _finish():
        dm = dmf_ref[...]
        dqf = dqf_ref[...]
        jio = jio_ref[...]
        den = jnp.float32(0.0)
        for _ in range(5):
            m = jnp.min(dm)
            sel = jnp.min(jnp.where(dm == m, jio, BIGI))
            oh = jio == sel
            dq = jnp.sum(jnp.where(oh, dqf, 0.0))
            den = den + jnp.exp(jnp.sqrt(jnp.clip(dq, 0.0)))
            dm = jnp.where(oh, jnp.inf, dm)

        dqb = jnp.sum(jnp.where(jio == b, dqf, 0.0))
        num = jnp.exp(jnp.sqrt(jnp.clip(dqb, 0.0)))
        s_star = sstar_ref[0, 0]
        s_ref[...] = jnp.full((8, 128), (1.0 - num / den) * s_star,
                              jnp.float32)

        v = jnp.sqrt(jnp.clip(v26_ref[...], 1e-12))
        rm = rmat_ref[...]
        bm = bmat_ref[...]
        t1 = lax.dot_general(rm, v, (((1,), (0,)), ((), ())),
                             preferred_element_type=jnp.float32)
        t2 = lax.dot_general(t1, rm, (((1,), (1,)), ((), ())),
                             preferred_element_type=jnp.float32)
        t3 = lax.dot_general(bm, t2, (((1,), (0,)), ((), ())),
                             preferred_element_type=jnp.float32)
        smap_ref[...] = lax.dot_general(t3, bm, (((1,), (1,)), ((), ())),
                                        preferred_element_type=jnp.float32)


def _run_kb(bidx1, patch_lib, mtest, v26, sstar, rmat, bmat, interpret=False):
    grid_spec = pltpu.PrefetchScalarGridSpec(
        num_scalar_prefetch=1,
        grid=(NSTEP,),
        in_specs=[
            pl.BlockSpec((BK, D), lambda k, b: (k, 0)),
            pl.BlockSpec((8, D), lambda k, b: (b[0] // 8, 0)),
            pl.BlockSpec((8, D), lambda k, b: (0, 0)),
            pl.BlockSpec((FMAP, FMAP), lambda k, b: (0, 0)),
            pl.BlockSpec((8, 128), lambda k, b: (0, 0)),
            pl.BlockSpec((IMG, FMAP), lambda k, b: (0, 0)),
            pl.BlockSpec((IMG, IMG), lambda k, b: (0, 0)),
        ],
        out_specs=[
            pl.BlockSpec((8, 128), lambda k, b: (0, 0)),
            pl.BlockSpec((IMG, IMG), lambda k, b: (0, 0)),
        ],
        scratch_shapes=[
            pltpu.VMEM((8, KF), jnp.float32),
            pltpu.VMEM((8, KF), jnp.float32),
            pltpu.VMEM((8, KF), jnp.int32),
        ],
    )
    return pl.pallas_call(
        _kb_body,
        grid_spec=grid_spec,
        out_shape=[
            jax.ShapeDtypeStruct((8, 128), jnp.float32),
            jax.ShapeDtypeStruct((IMG, IMG), jnp.float32),
        ],
        interpret=interpret,
    )(bidx1, patch_lib, patch_lib, mtest, v26, sstar, rmat, bmat)


def _blur_matrix():
    ax = np.arange(KSIZE, dtype=np.float32) - (KSIZE // 2)
    g = np.exp(-(ax ** 2) / (2.0 * SIGMA ** 2))
    g = g / np.sum(g)
    pad = KSIZE // 2
    eye = np.eye(IMG, dtype=np.float32)
    xp = np.pad(eye, ((pad, pad), (0, 0)), mode="reflect")
    b = np.zeros((IMG, IMG), dtype=np.float32)
    for t in range(KSIZE):
        b += g[t] * xp[t:t + IMG, :]
    return b


_BMAT = _blur_matrix()


def _resize_matrix():
    return jax.image.resize(jnp.eye(FMAP, dtype=jnp.float32), (IMG, FMAP),
                            method="bilinear")


def _kernel_impl(patch, patch_lib, interpret=False):
    patch_p = jnp.zeros((QP, D), jnp.float32).at[:Q].set(patch)
    mind2, bidx, mtest, sstar = _run_ka(patch_p, patch_lib, interpret)
    bidx1 = bidx[0, 0].reshape(1).astype(jnp.int32)
    v26 = mind2[0, :Q].reshape(FMAP, FMAP)
    s_out, smap = _run_kb(bidx1, patch_lib, mtest, v26, sstar,
                          _resize_matrix(), jnp.asarray(_BMAT), interpret)
    return s_out[0, 0], smap.reshape(1, 1, IMG, IMG)


def kernel(patch, patch_lib):
    return _kernel_impl(patch, patch_lib, interpret=False)
